# baseline (speedup 1.0000x reference)
import functools

import jax
import jax.numpy as jnp
from jax import lax
from jax.experimental import pallas as pl
from jax.experimental.pallas import tpu as pltpu
from jax.experimental.pallas import tpu_sc as plsc

_N = 10000
_E = 320000
_D = 128
_H = 256

_NW = 32
_K = 128
_CHK = 80
_HCHK = _CHK // 2
_E_PAD = _NW * _K * _CHK
_N_PAD = 10240
_RPS = _N_PAD // 16

_mesh = plsc.VectorSubcoreMesh(core_axis_name="c", subcore_axis_name="s")


@functools.partial(
    pl.kernel,
    out_type=jax.ShapeDtypeStruct((2, 2, _N_PAD), jnp.float32),
    mesh=_mesh,
    scratch_types=[
        pltpu.VMEM((_HCHK, _K), jnp.int32),
        pltpu.VMEM((_HCHK, _K), jnp.int32),
        pltpu.VMEM((_K,), jnp.float32),
        pltpu.VMEM((_K,), jnp.float32),
        pltpu.VMEM_SHARED((_N_PAD,), jnp.float32),
        pltpu.VMEM_SHARED((_N_PAD,), jnp.float32),
    ],
)
def _sc_degrees(src_hbm, dst_hbm, ones_hbm, zeros_hbm, out_hbm,
                src_v, dst_v, ones_v, zeros_v, dego_sh, degi_sh):
    c = lax.axis_index("c")
    s = lax.axis_index("s")
    pltpu.sync_copy(ones_hbm, ones_v)
    pltpu.sync_copy(zeros_hbm, zeros_v)
    @pl.loop(0, _RPS // _K)
    def _z(k):
        pltpu.sync_copy(zeros_v, dego_sh.at[pl.ds(s * _RPS + k * _K, _K)])
        pltpu.sync_copy(zeros_v, degi_sh.at[pl.ds(s * _RPS + k * _K, _K)])
    plsc.subcore_barrier()
    @pl.loop(0, 2)
    def _half(hh):
        pltpu.sync_copy(src_hbm.at[c].at[s].at[hh], src_v)
        pltpu.sync_copy(dst_hbm.at[c].at[s].at[hh], dst_v)
        @pl.loop(0, _HCHK)
        def _e(j):
            pltpu.sync_copy(ones_v, dego_sh.at[src_v.at[j]], add=True)
            pltpu.sync_copy(ones_v, degi_sh.at[dst_v.at[j]], add=True)
    plsc.subcore_barrier()
    pltpu.sync_copy(dego_sh.at[pl.ds(s * _RPS, _RPS)],
                    out_hbm.at[c].at[0].at[pl.ds(s * _RPS, _RPS)])
    pltpu.sync_copy(degi_sh.at[pl.ds(s * _RPS, _RPS)],
                    out_hbm.at[c].at[1].at[pl.ds(s * _RPS, _RPS)])


@functools.partial(
    pl.kernel,
    out_type=jax.ShapeDtypeStruct((2, _N_PAD, _D), jnp.float32),
    mesh=_mesh,
    scratch_types=[
        pltpu.VMEM((_HCHK, _K), jnp.int32),
        pltpu.VMEM((_HCHK, _K), jnp.int32),
        pltpu.VMEM((_K, _D), jnp.float32),
        pltpu.VMEM((_K, _D), jnp.float32),
        pltpu.VMEM((16, _D), jnp.float32),
        pltpu.VMEM_SHARED((_N_PAD, _D), jnp.float32),
        pltpu.SemaphoreType.DMA,
        pltpu.SemaphoreType.DMA,
    ],
)
def _sc_spmm(x_hbm, src_hbm, dst_hbm, z16_hbm, out_hbm,
             src_v, dst_v, rows0_v, rows1_v, zeros_v, agg_sh, sem0, sem1):
    c = lax.axis_index("c")
    s = lax.axis_index("s")
    pltpu.sync_copy(z16_hbm, zeros_v)
    @pl.loop(0, _RPS // 16)
    def _z(k):
        pltpu.sync_copy(zeros_v, agg_sh.at[pl.ds(s * _RPS + k * 16, 16)])
    plsc.subcore_barrier()
    @pl.loop(0, 2)
    def _half(hh):
        pltpu.sync_copy(src_hbm.at[c].at[s].at[hh], src_v)
        pltpu.sync_copy(dst_hbm.at[c].at[s].at[hh], dst_v)
        pltpu.async_copy(x_hbm.at[src_v.at[0]], rows0_v, sem0)
        pltpu.async_copy(x_hbm.at[src_v.at[1]], rows1_v, sem1)
        @pl.loop(0, _HCHK, step=2)
        def _e(j):
            pltpu.make_async_copy(x_hbm.at[src_v.at[j]], rows0_v, sem0).wait()
            pltpu.sync_copy(rows0_v, agg_sh.at[dst_v.at[j]], add=True)
            @pl.when(j + 2 < _HCHK)
            def _():
                pltpu.async_copy(x_hbm.at[src_v.at[j + 2]], rows0_v, sem0)
            pltpu.make_async_copy(x_hbm.at[src_v.at[j + 1]], rows1_v, sem1).wait()
            pltpu.sync_copy(rows1_v, agg_sh.at[dst_v.at[j + 1]], add=True)
            @pl.when(j + 3 < _HCHK)
            def _():
                pltpu.async_copy(x_hbm.at[src_v.at[j + 3]], rows1_v, sem1)
    plsc.subcore_barrier()
    pltpu.sync_copy(agg_sh.at[pl.ds(s * _RPS, _RPS)],
                    out_hbm.at[c].at[pl.ds(s * _RPS, _RPS)])


_R = 1024


def _prep_body(deg_ref, h_ref, invout_ref, invin_ref, xs_ref):
    d = deg_ref[...]
    io = lax.rsqrt(jnp.maximum(d[0, 0] + d[1, 0], 1.0))[:, None]
    ii = lax.rsqrt(jnp.maximum(d[0, 1] + d[1, 1], 1.0))[:, None]
    invout_ref[...] = io
    invin_ref[...] = ii
    xs_ref[...] = h_ref[...] * io


def _mid_body(pa_ref, invin_ref, invout_ref, w1_ref, b1_ref, h1a_ref, h1b_ref):
    agg = (pa_ref[0] + pa_ref[1]) * invin_ref[...]
    y = jnp.dot(agg, w1_ref[...], preferred_element_type=jnp.float32)
    y = jnp.maximum(y + b1_ref[...], 0.0) * invout_ref[...]
    h1a_ref[...] = y[:, :_D]
    h1b_ref[...] = y[:, _D:]


def _fin_body(pa_ref, pb_ref, invin_ref, w2a_ref, w2b_ref, b2_ref,
              wc_ref, bc_ref, out_ref, pool_ref):
    i = pl.program_id(0)
    ii = invin_ref[...]
    agg_a = (pa_ref[0] + pa_ref[1]) * ii
    agg_b = (pb_ref[0] + pb_ref[1]) * ii
    y = (jnp.dot(agg_a, w2a_ref[...], preferred_element_type=jnp.float32)
         + jnp.dot(agg_b, w2b_ref[...], preferred_element_type=jnp.float32))
    y = jnp.maximum(y + b2_ref[...], 0.0)
    rows = lax.broadcasted_iota(jnp.int32, (_R, 1), 0) + i * _R
    y = jnp.where(rows < _N, y, 0.0)
    bm = jnp.max(y, axis=0, keepdims=True)
    @pl.when(i == 0)
    def _():
        pool_ref[...] = bm
    @pl.when(i > 0)
    def _():
        pool_ref[...] = jnp.maximum(pool_ref[...], bm)
    @pl.when(i == pl.num_programs(0) - 1)
    def _():
        out_ref[...] = (jnp.dot(pool_ref[...], wc_ref[...],
                                preferred_element_type=jnp.float32)
                        + bc_ref[...])


def _tc_prep(deg_parts, h_pad):
    g = _N_PAD // _R
    return pl.pallas_call(
        _prep_body,
        grid=(g,),
        in_specs=[
            pl.BlockSpec((2, 2, _R), lambda i: (0, 0, i)),
            pl.BlockSpec((_R, _D), lambda i: (i, 0)),
        ],
        out_specs=[
            pl.BlockSpec((_R, 1), lambda i: (i, 0)),
            pl.BlockSpec((_R, 1), lambda i: (i, 0)),
            pl.BlockSpec((_R, _D), lambda i: (i, 0)),
        ],
        out_shape=[
            jax.ShapeDtypeStruct((_N_PAD, 1), jnp.float32),
            jax.ShapeDtypeStruct((_N_PAD, 1), jnp.float32),
            jax.ShapeDtypeStruct((_N_PAD, _D), jnp.float32),
        ],
    )(deg_parts, h_pad)


def _tc_mid(parts, invin, invout, w1, b1r):
    g = _N_PAD // _R
    return pl.pallas_call(
        _mid_body,
        grid=(g,),
        in_specs=[
            pl.BlockSpec((2, _R, _D), lambda i: (0, i, 0)),
            pl.BlockSpec((_R, 1), lambda i: (i, 0)),
            pl.BlockSpec((_R, 1), lambda i: (i, 0)),
            pl.BlockSpec((_D, _H), lambda i: (0, 0)),
            pl.BlockSpec((1, _H), lambda i: (0, 0)),
        ],
        out_specs=[
            pl.BlockSpec((_R, _D), lambda i: (i, 0)),
            pl.BlockSpec((_R, _D), lambda i: (i, 0)),
        ],
        out_shape=[
            jax.ShapeDtypeStruct((_N_PAD, _D), jnp.float32),
            jax.ShapeDtypeStruct((_N_PAD, _D), jnp.float32),
        ],
    )(parts, invin, invout, w1, b1r)


def _tc_final(parts_a, parts_b, invin, w2a, w2b, b2r, wc_pad, bc_pad):
    g = _N_PAD // _R
    return pl.pallas_call(
        _fin_body,
        grid=(g,),
        in_specs=[
            pl.BlockSpec((2, _R, _D), lambda i: (0, i, 0)),
            pl.BlockSpec((2, _R, _D), lambda i: (0, i, 0)),
            pl.BlockSpec((_R, 1), lambda i: (i, 0)),
            pl.BlockSpec((_D, _H), lambda i: (0, 0)),
            pl.BlockSpec((_D, _H), lambda i: (0, 0)),
            pl.BlockSpec((1, _H), lambda i: (0, 0)),
            pl.BlockSpec((_H, 128), lambda i: (0, 0)),
            pl.BlockSpec((1, 128), lambda i: (0, 0)),
        ],
        out_specs=pl.BlockSpec((1, 128), lambda i: (0, 0)),
        out_shape=jax.ShapeDtypeStruct((1, 128), jnp.float32),
        scratch_shapes=[pltpu.VMEM((1, _H), jnp.float32)],
    )(parts_a, parts_b, invin, w2a, w2b, b2r, wc_pad, bc_pad)


def kernel(h, edge_index, W1, b1, W2, b2, Wc, bc):
    src = edge_index[0]
    dst = edge_index[1]
    pad = jnp.full((_E_PAD - _E,), _N_PAD - 1, dtype=jnp.int32)
    srcp = jnp.concatenate([src, pad]).reshape(2, 16, 2, _HCHK, _K)
    dstp = jnp.concatenate([dst, pad]).reshape(2, 16, 2, _HCHK, _K)

    ones128 = jnp.ones((_K,), jnp.float32)
    zeros128 = jnp.zeros((_K,), jnp.float32)
    z16 = jnp.zeros((16, _D), jnp.float32)

    deg_parts = _sc_degrees(srcp, dstp, ones128, zeros128)

    h_pad = jnp.pad(h, ((0, _N_PAD - _N), (0, 0)))
    invout, invin, xs = _tc_prep(deg_parts, h_pad)

    agg1_parts = _sc_spmm(xs, srcp, dstp, z16)

    h1a, h1b = _tc_mid(agg1_parts, invin, invout, W1, b1.reshape(1, _H))

    agg2a_parts = _sc_spmm(h1a, srcp, dstp, z16)
    agg2b_parts = _sc_spmm(h1b, srcp, dstp, z16)

    wc_pad = jnp.pad(Wc, ((0, 0), (0, 128 - Wc.shape[1])))
    bc_pad = jnp.pad(bc, (0, 128 - bc.shape[0])).reshape(1, 128)
    out = _tc_final(agg2a_parts, agg2b_parts, invin,
                    W2[:_D], W2[_D:], b2.reshape(1, _H), wc_pad, bc_pad)
    return out[0, :Wc.shape[1]]


# speedup vs baseline: 2.8115x; 2.8115x over previous
import functools

import jax
import jax.numpy as jnp
from jax import lax
from jax.experimental import pallas as pl
from jax.experimental.pallas import tpu as pltpu
from jax.experimental.pallas import tpu_sc as plsc

_N = 10000
_E = 320000
_D = 128
_H = 256

_NW = 32
_K = 128
_CHK = 80
_HCHK = _CHK // 2
_E_PAD = _NW * _K * _CHK
_N_PAD = 10240
_RPS = _N_PAD // 16

_mesh = plsc.VectorSubcoreMesh(core_axis_name="c", subcore_axis_name="s")


@functools.partial(
    pl.kernel,
    out_type=jax.ShapeDtypeStruct((2, 2, _N_PAD), jnp.float32),
    mesh=_mesh,
    scratch_types=[
        pltpu.VMEM((2, _HCHK, _K), jnp.int32),
        pltpu.VMEM((2, _HCHK, _K), jnp.int32),
        pltpu.VMEM((_K,), jnp.float32),
        pltpu.VMEM((_K,), jnp.float32),
        pltpu.VMEM_SHARED((_N_PAD,), jnp.float32),
        pltpu.VMEM_SHARED((_N_PAD,), jnp.float32),
        pltpu.SemaphoreType.DMA,
    ],
)
def _sc_degrees(src_hbm, dst_hbm, ones_hbm, zeros_hbm, out_hbm,
                src_v, dst_v, ones_v, zeros_v, dego_sh, degi_sh, dsem):
    c = lax.axis_index("c")
    s = lax.axis_index("s")
    pltpu.sync_copy(ones_hbm, ones_v)
    pltpu.sync_copy(zeros_hbm, zeros_v)
    @pl.loop(0, _RPS // _K)
    def _z(k):
        pltpu.sync_copy(zeros_v, dego_sh.at[pl.ds(s * _RPS + k * _K, _K)])
        pltpu.sync_copy(zeros_v, degi_sh.at[pl.ds(s * _RPS + k * _K, _K)])
    plsc.subcore_barrier()
    pltpu.sync_copy(src_hbm.at[c].at[s], src_v)
    pltpu.sync_copy(dst_hbm.at[c].at[s], dst_v)
    @pl.loop(0, 2)
    def _half(hh):
        @pl.loop(0, _HCHK)
        def _e(j):
            pltpu.async_copy(ones_v, dego_sh.at[src_v.at[hh].at[j]], dsem,
                             add=True)
            pltpu.async_copy(ones_v, degi_sh.at[dst_v.at[hh].at[j]], dsem,
                             add=True)
    @pl.loop(0, 2 * _CHK // _HCHK)
    def _drain(k):
        pltpu.make_async_copy(src_hbm.at[c].at[s].at[0], src_v.at[0],
                              dsem).wait()
    plsc.subcore_barrier()
    pltpu.sync_copy(dego_sh.at[pl.ds(s * _RPS, _RPS)],
                    out_hbm.at[c].at[0].at[pl.ds(s * _RPS, _RPS)])
    pltpu.sync_copy(degi_sh.at[pl.ds(s * _RPS, _RPS)],
                    out_hbm.at[c].at[1].at[pl.ds(s * _RPS, _RPS)])


@functools.partial(
    pl.kernel,
    out_type=jax.ShapeDtypeStruct((2, _N_PAD, _D), jnp.float32),
    mesh=_mesh,
    scratch_types=[
        pltpu.VMEM((_HCHK, _K), jnp.int32),
        pltpu.VMEM((_HCHK, _K), jnp.int32),
        pltpu.VMEM((_K, _D), jnp.float32),
        pltpu.VMEM((_K, _D), jnp.float32),
        pltpu.VMEM((16, _D), jnp.float32),
        pltpu.VMEM_SHARED((_N_PAD, _D), jnp.float32),
        pltpu.SemaphoreType.DMA,
        pltpu.SemaphoreType.DMA,
        pltpu.SemaphoreType.DMA,
        pltpu.SemaphoreType.DMA,
    ],
)
def _sc_spmm(x_hbm, src_hbm, dst_hbm, z16_hbm, out_hbm,
             src_v, dst_v, rows0_v, rows1_v, zeros_v, agg_sh,
             gsem0, gsem1, ssem0, ssem1):
    c = lax.axis_index("c")
    s = lax.axis_index("s")
    pltpu.sync_copy(z16_hbm, zeros_v)
    @pl.loop(0, _RPS // 16)
    def _z(k):
        pltpu.sync_copy(zeros_v, agg_sh.at[pl.ds(s * _RPS + k * 16, 16)])
    plsc.subcore_barrier()
    @pl.loop(0, 2)
    def _half(hh):
        pltpu.sync_copy(src_hbm.at[c].at[s].at[hh], src_v)
        pltpu.sync_copy(dst_hbm.at[c].at[s].at[hh], dst_v)
        pltpu.async_copy(x_hbm.at[src_v.at[0]], rows0_v, gsem0)
        pltpu.async_copy(x_hbm.at[src_v.at[1]], rows1_v, gsem1)
        @pl.loop(0, _HCHK, step=2)
        def _e(j):
            pltpu.make_async_copy(x_hbm.at[src_v.at[j]], rows0_v, gsem0).wait()
            pltpu.async_copy(rows0_v, agg_sh.at[dst_v.at[j]], ssem0, add=True)
            pltpu.make_async_copy(x_hbm.at[src_v.at[j + 1]], rows1_v, gsem1).wait()
            pltpu.async_copy(rows1_v, agg_sh.at[dst_v.at[j + 1]], ssem1, add=True)
            pltpu.make_async_copy(rows0_v, agg_sh.at[dst_v.at[j]], ssem0).wait()
            @pl.when(j + 2 < _HCHK)
            def _():
                pltpu.async_copy(x_hbm.at[src_v.at[j + 2]], rows0_v, gsem0)
            pltpu.make_async_copy(rows1_v, agg_sh.at[dst_v.at[j + 1]], ssem1).wait()
            @pl.when(j + 3 < _HCHK)
            def _():
                pltpu.async_copy(x_hbm.at[src_v.at[j + 3]], rows1_v, gsem1)
    plsc.subcore_barrier()
    pltpu.sync_copy(agg_sh.at[pl.ds(s * _RPS, _RPS)],
                    out_hbm.at[c].at[pl.ds(s * _RPS, _RPS)])


_R = 1024


def _prep_body(deg_ref, h_ref, invout_ref, invin_ref, xs_ref):
    d = deg_ref[...]
    io = lax.rsqrt(jnp.maximum(d[0, 0] + d[1, 0], 1.0))[:, None]
    ii = lax.rsqrt(jnp.maximum(d[0, 1] + d[1, 1], 1.0))[:, None]
    invout_ref[...] = io
    invin_ref[...] = ii
    xs_ref[...] = h_ref[...] * io


def _mid_body(pa_ref, invin_ref, invout_ref, w1_ref, b1_ref, h1a_ref, h1b_ref):
    agg = (pa_ref[0] + pa_ref[1]) * invin_ref[...]
    y = jnp.dot(agg, w1_ref[...], preferred_element_type=jnp.float32)
    y = jnp.maximum(y + b1_ref[...], 0.0) * invout_ref[...]
    h1a_ref[...] = y[:, :_D]
    h1b_ref[...] = y[:, _D:]


def _fin_body(pa_ref, pb_ref, invin_ref, w2a_ref, w2b_ref, b2_ref,
              wc_ref, bc_ref, out_ref, pool_ref):
    i = pl.program_id(0)
    ii = invin_ref[...]
    agg_a = (pa_ref[0] + pa_ref[1]) * ii
    agg_b = (pb_ref[0] + pb_ref[1]) * ii
    y = (jnp.dot(agg_a, w2a_ref[...], preferred_element_type=jnp.float32)
         + jnp.dot(agg_b, w2b_ref[...], preferred_element_type=jnp.float32))
    y = jnp.maximum(y + b2_ref[...], 0.0)
    rows = lax.broadcasted_iota(jnp.int32, (_R, 1), 0) + i * _R
    y = jnp.where(rows < _N, y, 0.0)
    bm = jnp.max(y, axis=0, keepdims=True)
    @pl.when(i == 0)
    def _():
        pool_ref[...] = bm
    @pl.when(i > 0)
    def _():
        pool_ref[...] = jnp.maximum(pool_ref[...], bm)
    @pl.when(i == pl.num_programs(0) - 1)
    def _():
        out_ref[...] = (jnp.dot(pool_ref[...], wc_ref[...],
                                preferred_element_type=jnp.float32)
                        + bc_ref[...])


def _tc_prep(deg_parts, h_pad):
    g = _N_PAD // _R
    return pl.pallas_call(
        _prep_body,
        grid=(g,),
        in_specs=[
            pl.BlockSpec((2, 2, _R), lambda i: (0, 0, i)),
            pl.BlockSpec((_R, _D), lambda i: (i, 0)),
        ],
        out_specs=[
            pl.BlockSpec((_R, 1), lambda i: (i, 0)),
            pl.BlockSpec((_R, 1), lambda i: (i, 0)),
            pl.BlockSpec((_R, _D), lambda i: (i, 0)),
        ],
        out_shape=[
            jax.ShapeDtypeStruct((_N_PAD, 1), jnp.float32),
            jax.ShapeDtypeStruct((_N_PAD, 1), jnp.float32),
            jax.ShapeDtypeStruct((_N_PAD, _D), jnp.float32),
        ],
    )(deg_parts, h_pad)


def _tc_mid(parts, invin, invout, w1, b1r):
    g = _N_PAD // _R
    return pl.pallas_call(
        _mid_body,
        grid=(g,),
        in_specs=[
            pl.BlockSpec((2, _R, _D), lambda i: (0, i, 0)),
            pl.BlockSpec((_R, 1), lambda i: (i, 0)),
            pl.BlockSpec((_R, 1), lambda i: (i, 0)),
            pl.BlockSpec((_D, _H), lambda i: (0, 0)),
            pl.BlockSpec((1, _H), lambda i: (0, 0)),
        ],
        out_specs=[
            pl.BlockSpec((_R, _D), lambda i: (i, 0)),
            pl.BlockSpec((_R, _D), lambda i: (i, 0)),
        ],
        out_shape=[
            jax.ShapeDtypeStruct((_N_PAD, _D), jnp.float32),
            jax.ShapeDtypeStruct((_N_PAD, _D), jnp.float32),
        ],
    )(parts, invin, invout, w1, b1r)


def _tc_final(parts_a, parts_b, invin, w2a, w2b, b2r, wc_pad, bc_pad):
    g = _N_PAD // _R
    return pl.pallas_call(
        _fin_body,
        grid=(g,),
        in_specs=[
            pl.BlockSpec((2, _R, _D), lambda i: (0, i, 0)),
            pl.BlockSpec((2, _R, _D), lambda i: (0, i, 0)),
            pl.BlockSpec((_R, 1), lambda i: (i, 0)),
            pl.BlockSpec((_D, _H), lambda i: (0, 0)),
            pl.BlockSpec((_D, _H), lambda i: (0, 0)),
            pl.BlockSpec((1, _H), lambda i: (0, 0)),
            pl.BlockSpec((_H, 128), lambda i: (0, 0)),
            pl.BlockSpec((1, 128), lambda i: (0, 0)),
        ],
        out_specs=pl.BlockSpec((1, 128), lambda i: (0, 0)),
        out_shape=jax.ShapeDtypeStruct((1, 128), jnp.float32),
        scratch_shapes=[pltpu.VMEM((1, _H), jnp.float32)],
    )(parts_a, parts_b, invin, w2a, w2b, b2r, wc_pad, bc_pad)


def kernel(h, edge_index, W1, b1, W2, b2, Wc, bc):
    src = edge_index[0]
    dst = edge_index[1]
    ppt = _E_PAD // _NW - _E // _NW
    pads = jnp.broadcast_to(_N + jnp.arange(ppt, dtype=jnp.int32), (_NW, ppt))
    srcp = jnp.concatenate([src.reshape(_NW, _E // _NW), pads],
                           axis=1).reshape(2, 16, 2, _HCHK, _K)
    dstp = jnp.concatenate([dst.reshape(_NW, _E // _NW), pads],
                           axis=1).reshape(2, 16, 2, _HCHK, _K)

    ones128 = jnp.ones((_K,), jnp.float32)
    zeros128 = jnp.zeros((_K,), jnp.float32)
    z16 = jnp.zeros((16, _D), jnp.float32)

    deg_parts = _sc_degrees(srcp, dstp, ones128, zeros128)

    h_pad = jnp.pad(h, ((0, _N_PAD - _N), (0, 0)))
    invout, invin, xs = _tc_prep(deg_parts, h_pad)

    agg1_parts = _sc_spmm(xs, srcp, dstp, z16)

    h1a, h1b = _tc_mid(agg1_parts, invin, invout, W1, b1.reshape(1, _H))

    agg2a_parts = _sc_spmm(h1a, srcp, dstp, z16)
    agg2b_parts = _sc_spmm(h1b, srcp, dstp, z16)

    wc_pad = jnp.pad(Wc, ((0, 0), (0, 128 - Wc.shape[1])))
    bc_pad = jnp.pad(bc, (0, 128 - bc.shape[0])).reshape(1, 128)
    out = _tc_final(agg2a_parts, agg2b_parts, invin,
                    W2[:_D], W2[_D:], b2.reshape(1, _H), wc_pad, bc_pad)
    return out[0, :Wc.shape[1]]


# speedup vs baseline: 3.5411x; 1.2595x over previous
import functools

import jax
import jax.numpy as jnp
from jax import lax
from jax.experimental import pallas as pl
from jax.experimental.pallas import tpu as pltpu
from jax.experimental.pallas import tpu_sc as plsc

_N = 10000
_E = 320000
_D = 128
_H = 256

_NW = 32
_K = 128
_CHK = 80
_HCHK = _CHK // 2
_E_PAD = _NW * _K * _CHK
_N_PAD = 10240
_RPS = _N_PAD // 16

_mesh = plsc.VectorSubcoreMesh(core_axis_name="c", subcore_axis_name="s")


@functools.partial(
    pl.kernel,
    out_type=jax.ShapeDtypeStruct((2, 2, _N_PAD), jnp.float32),
    mesh=_mesh,
    scratch_types=[
        pltpu.VMEM((2, _HCHK, _K), jnp.int32),
        pltpu.VMEM((2, _HCHK, _K), jnp.int32),
        pltpu.VMEM((_K,), jnp.float32),
        pltpu.VMEM((_K,), jnp.float32),
        pltpu.VMEM_SHARED((_N_PAD,), jnp.float32),
        pltpu.VMEM_SHARED((_N_PAD,), jnp.float32),
        pltpu.SemaphoreType.DMA,
    ],
)
def _sc_degrees(src_hbm, dst_hbm, ones_hbm, zeros_hbm, out_hbm,
                src_v, dst_v, ones_v, zeros_v, dego_sh, degi_sh, dsem):
    c = lax.axis_index("c")
    s = lax.axis_index("s")
    pltpu.sync_copy(ones_hbm, ones_v)
    pltpu.sync_copy(zeros_hbm, zeros_v)
    @pl.loop(0, _RPS // _K)
    def _z(k):
        pltpu.sync_copy(zeros_v, dego_sh.at[pl.ds(s * _RPS + k * _K, _K)])
        pltpu.sync_copy(zeros_v, degi_sh.at[pl.ds(s * _RPS + k * _K, _K)])
    plsc.subcore_barrier()
    pltpu.sync_copy(src_hbm.at[c].at[s], src_v)
    pltpu.sync_copy(dst_hbm.at[c].at[s], dst_v)
    @pl.loop(0, 2)
    def _half(hh):
        @pl.loop(0, _HCHK)
        def _e(j):
            pltpu.async_copy(ones_v, dego_sh.at[src_v.at[hh].at[j]], dsem,
                             add=True)
            pltpu.async_copy(ones_v, degi_sh.at[dst_v.at[hh].at[j]], dsem,
                             add=True)
    @pl.loop(0, 2 * _CHK // _HCHK)
    def _drain(k):
        pltpu.make_async_copy(src_hbm.at[c].at[s].at[0], src_v.at[0],
                              dsem).wait()
    plsc.subcore_barrier()
    pltpu.sync_copy(dego_sh.at[pl.ds(s * _RPS, _RPS)],
                    out_hbm.at[c].at[0].at[pl.ds(s * _RPS, _RPS)])
    pltpu.sync_copy(degi_sh.at[pl.ds(s * _RPS, _RPS)],
                    out_hbm.at[c].at[1].at[pl.ds(s * _RPS, _RPS)])


@functools.partial(
    pl.kernel,
    out_type=jax.ShapeDtypeStruct((2, _N_PAD, _D), jnp.float32),
    mesh=_mesh,
    scratch_types=[
        pltpu.VMEM((_HCHK, _K), jnp.int32),
        pltpu.VMEM((_HCHK, _K), jnp.int32),
        pltpu.VMEM((_K, _D), jnp.float32),
        pltpu.VMEM((_K, _D), jnp.float32),
        pltpu.VMEM((16, _D), jnp.float32),
        pltpu.VMEM_SHARED((_N_PAD, _D), jnp.float32),
        pltpu.SemaphoreType.DMA,
        pltpu.SemaphoreType.DMA,
    ],
)
def _sc_spmm(x_hbm, src_hbm, dst_hbm, z16_hbm, out_hbm,
             src_v, dst_v, rows0_v, rows1_v, zeros_v, agg_sh,
             gsem0, gsem1):
    c = lax.axis_index("c")
    s = lax.axis_index("s")
    pltpu.sync_copy(z16_hbm, zeros_v)
    @pl.loop(0, _RPS // 16)
    def _z(k):
        pltpu.sync_copy(zeros_v, agg_sh.at[pl.ds(s * _RPS + k * 16, 16)])
    plsc.subcore_barrier()
    @pl.loop(0, 2)
    def _half(hh):
        pltpu.sync_copy(src_hbm.at[c].at[s].at[hh], src_v)
        pltpu.sync_copy(dst_hbm.at[c].at[s].at[hh], dst_v)
        pltpu.async_copy(x_hbm.at[src_v.at[0]], rows0_v, gsem0)
        pltpu.async_copy(x_hbm.at[src_v.at[1]], rows1_v, gsem1)
        @pl.loop(0, _HCHK, step=2)
        def _e(j):
            pltpu.make_async_copy(x_hbm.at[src_v.at[j]], rows0_v, gsem0).wait()
            pltpu.sync_copy(rows0_v, agg_sh.at[dst_v.at[j]], add=True)
            @pl.when(j + 2 < _HCHK)
            def _():
                pltpu.async_copy(x_hbm.at[src_v.at[j + 2]], rows0_v, gsem0)
            pltpu.make_async_copy(x_hbm.at[src_v.at[j + 1]], rows1_v, gsem1).wait()
            pltpu.sync_copy(rows1_v, agg_sh.at[dst_v.at[j + 1]], add=True)
            @pl.when(j + 3 < _HCHK)
            def _():
                pltpu.async_copy(x_hbm.at[src_v.at[j + 3]], rows1_v, gsem1)
    plsc.subcore_barrier()
    pltpu.sync_copy(agg_sh.at[pl.ds(s * _RPS, _RPS)],
                    out_hbm.at[c].at[pl.ds(s * _RPS, _RPS)])


_R = 1024


def _prep_body(deg_ref, h_ref, invout_ref, invin_ref, xs_ref):
    d = deg_ref[...]
    io = lax.rsqrt(jnp.maximum(d[0, 0] + d[1, 0], 1.0))[:, None]
    ii = lax.rsqrt(jnp.maximum(d[0, 1] + d[1, 1], 1.0))[:, None]
    invout_ref[...] = io
    invin_ref[...] = ii
    xs_ref[...] = h_ref[...] * io


def _mid_body(pa_ref, invin_ref, invout_ref, w1_ref, b1_ref, h1a_ref, h1b_ref):
    agg = (pa_ref[0] + pa_ref[1]) * invin_ref[...]
    y = jnp.dot(agg, w1_ref[...], preferred_element_type=jnp.float32)
    y = jnp.maximum(y + b1_ref[...], 0.0) * invout_ref[...]
    h1a_ref[...] = y[:, :_D]
    h1b_ref[...] = y[:, _D:]


def _fin_body(pa_ref, pb_ref, invin_ref, w2a_ref, w2b_ref, b2_ref,
              wc_ref, bc_ref, out_ref, pool_ref):
    i = pl.program_id(0)
    ii = invin_ref[...]
    agg_a = (pa_ref[0] + pa_ref[1]) * ii
    agg_b = (pb_ref[0] + pb_ref[1]) * ii
    y = (jnp.dot(agg_a, w2a_ref[...], preferred_element_type=jnp.float32)
         + jnp.dot(agg_b, w2b_ref[...], preferred_element_type=jnp.float32))
    y = jnp.maximum(y + b2_ref[...], 0.0)
    rows = lax.broadcasted_iota(jnp.int32, (_R, 1), 0) + i * _R
    y = jnp.where(rows < _N, y, 0.0)
    bm = jnp.max(y, axis=0, keepdims=True)
    @pl.when(i == 0)
    def _():
        pool_ref[...] = bm
    @pl.when(i > 0)
    def _():
        pool_ref[...] = jnp.maximum(pool_ref[...], bm)
    @pl.when(i == pl.num_programs(0) - 1)
    def _():
        out_ref[...] = (jnp.dot(pool_ref[...], wc_ref[...],
                                preferred_element_type=jnp.float32)
                        + bc_ref[...])


def _tc_prep(deg_parts, h_pad):
    g = _N_PAD // _R
    return pl.pallas_call(
        _prep_body,
        grid=(g,),
        in_specs=[
            pl.BlockSpec((2, 2, _R), lambda i: (0, 0, i)),
            pl.BlockSpec((_R, _D), lambda i: (i, 0)),
        ],
        out_specs=[
            pl.BlockSpec((_R, 1), lambda i: (i, 0)),
            pl.BlockSpec((_R, 1), lambda i: (i, 0)),
            pl.BlockSpec((_R, _D), lambda i: (i, 0)),
        ],
        out_shape=[
            jax.ShapeDtypeStruct((_N_PAD, 1), jnp.float32),
            jax.ShapeDtypeStruct((_N_PAD, 1), jnp.float32),
            jax.ShapeDtypeStruct((_N_PAD, _D), jnp.float32),
        ],
    )(deg_parts, h_pad)


def _tc_mid(parts, invin, invout, w1, b1r):
    g = _N_PAD // _R
    return pl.pallas_call(
        _mid_body,
        grid=(g,),
        in_specs=[
            pl.BlockSpec((2, _R, _D), lambda i: (0, i, 0)),
            pl.BlockSpec((_R, 1), lambda i: (i, 0)),
            pl.BlockSpec((_R, 1), lambda i: (i, 0)),
            pl.BlockSpec((_D, _H), lambda i: (0, 0)),
            pl.BlockSpec((1, _H), lambda i: (0, 0)),
        ],
        out_specs=[
            pl.BlockSpec((_R, _D), lambda i: (i, 0)),
            pl.BlockSpec((_R, _D), lambda i: (i, 0)),
        ],
        out_shape=[
            jax.ShapeDtypeStruct((_N_PAD, _D), jnp.float32),
            jax.ShapeDtypeStruct((_N_PAD, _D), jnp.float32),
        ],
    )(parts, invin, invout, w1, b1r)


def _tc_final(parts_a, parts_b, invin, w2a, w2b, b2r, wc_pad, bc_pad):
    g = _N_PAD // _R
    return pl.pallas_call(
        _fin_body,
        grid=(g,),
        in_specs=[
            pl.BlockSpec((2, _R, _D), lambda i: (0, i, 0)),
            pl.BlockSpec((2, _R, _D), lambda i: (0, i, 0)),
            pl.BlockSpec((_R, 1), lambda i: (i, 0)),
            pl.BlockSpec((_D, _H), lambda i: (0, 0)),
            pl.BlockSpec((_D, _H), lambda i: (0, 0)),
            pl.BlockSpec((1, _H), lambda i: (0, 0)),
            pl.BlockSpec((_H, 128), lambda i: (0, 0)),
            pl.BlockSpec((1, 128), lambda i: (0, 0)),
        ],
        out_specs=pl.BlockSpec((1, 128), lambda i: (0, 0)),
        out_shape=jax.ShapeDtypeStruct((1, 128), jnp.float32),
        scratch_shapes=[pltpu.VMEM((1, _H), jnp.float32)],
    )(parts_a, parts_b, invin, w2a, w2b, b2r, wc_pad, bc_pad)


def kernel(h, edge_index, W1, b1, W2, b2, Wc, bc):
    src = edge_index[0]
    dst = edge_index[1]
    ppt = _E_PAD // _NW - _E // _NW
    pads = jnp.broadcast_to(_N + jnp.arange(ppt, dtype=jnp.int32), (_NW, ppt))
    srcp = jnp.concatenate([src.reshape(_NW, _E // _NW), pads],
                           axis=1).reshape(2, 16, 2, _HCHK, _K)
    dstp = jnp.concatenate([dst.reshape(_NW, _E // _NW), pads],
                           axis=1).reshape(2, 16, 2, _HCHK, _K)

    ones128 = jnp.ones((_K,), jnp.float32)
    zeros128 = jnp.zeros((_K,), jnp.float32)
    z16 = jnp.zeros((16, _D), jnp.float32)

    deg_parts = _sc_degrees(srcp, dstp, ones128, zeros128)

    h_pad = jnp.pad(h, ((0, _N_PAD - _N), (0, 0)))
    invout, invin, xs = _tc_prep(deg_parts, h_pad)

    agg1_parts = _sc_spmm(xs, srcp, dstp, z16)

    h1a, h1b = _tc_mid(agg1_parts, invin, invout, W1, b1.reshape(1, _H))

    agg2a_parts = _sc_spmm(h1a, srcp, dstp, z16)
    agg2b_parts = _sc_spmm(h1b, srcp, dstp, z16)

    wc_pad = jnp.pad(Wc, ((0, 0), (0, 128 - Wc.shape[1])))
    bc_pad = jnp.pad(bc, (0, 128 - bc.shape[0])).reshape(1, 128)
    out = _tc_final(agg2a_parts, agg2b_parts, invin,
                    W2[:_D], W2[_D:], b2.reshape(1, _H), wc_pad, bc_pad)
    return out[0, :Wc.shape[1]]


# speedup vs baseline: 3.7262x; 1.0523x over previous
import functools

import jax
import jax.numpy as jnp
from jax import lax
from jax.experimental import pallas as pl
from jax.experimental.pallas import tpu as pltpu
from jax.experimental.pallas import tpu_sc as plsc

_N = 10000
_E = 320000
_D = 128
_H = 256

_NW = 32
_K = 128
_CHK = 80
_HCHK = _CHK // 2
_E_PAD = _NW * _K * _CHK
_N_PAD = 10240
_RPS = _N_PAD // 16

_mesh = plsc.VectorSubcoreMesh(core_axis_name="c", subcore_axis_name="s")


@functools.partial(
    pl.kernel,
    out_type=jax.ShapeDtypeStruct((2, 2, _N_PAD), jnp.float32),
    mesh=_mesh,
    scratch_types=[
        pltpu.VMEM((2, _HCHK, _K), jnp.int32),
        pltpu.VMEM((2, _HCHK, _K), jnp.int32),
        pltpu.VMEM((_K,), jnp.float32),
        pltpu.VMEM((_K,), jnp.float32),
        pltpu.VMEM_SHARED((_N_PAD,), jnp.float32),
        pltpu.VMEM_SHARED((_N_PAD,), jnp.float32),
        pltpu.SemaphoreType.DMA,
    ],
)
def _sc_degrees(src_hbm, dst_hbm, ones_hbm, zeros_hbm, out_hbm,
                src_v, dst_v, ones_v, zeros_v, dego_sh, degi_sh, dsem):
    c = lax.axis_index("c")
    s = lax.axis_index("s")
    pltpu.sync_copy(ones_hbm, ones_v)
    pltpu.sync_copy(zeros_hbm, zeros_v)
    @pl.loop(0, _RPS // _K)
    def _z(k):
        pltpu.sync_copy(zeros_v, dego_sh.at[pl.ds(s * _RPS + k * _K, _K)])
        pltpu.sync_copy(zeros_v, degi_sh.at[pl.ds(s * _RPS + k * _K, _K)])
    plsc.subcore_barrier()
    pltpu.sync_copy(src_hbm.at[c].at[s], src_v)
    pltpu.sync_copy(dst_hbm.at[c].at[s], dst_v)
    @pl.loop(0, 2)
    def _half(hh):
        @pl.loop(0, _HCHK)
        def _e(j):
            pltpu.async_copy(ones_v, dego_sh.at[src_v.at[hh].at[j]], dsem,
                             add=True)
            pltpu.async_copy(ones_v, degi_sh.at[dst_v.at[hh].at[j]], dsem,
                             add=True)
    @pl.loop(0, 2 * _CHK // _HCHK)
    def _drain(k):
        pltpu.make_async_copy(src_hbm.at[c].at[s].at[0], src_v.at[0],
                              dsem).wait()
    plsc.subcore_barrier()
    pltpu.sync_copy(dego_sh.at[pl.ds(s * _RPS, _RPS)],
                    out_hbm.at[c].at[0].at[pl.ds(s * _RPS, _RPS)])
    pltpu.sync_copy(degi_sh.at[pl.ds(s * _RPS, _RPS)],
                    out_hbm.at[c].at[1].at[pl.ds(s * _RPS, _RPS)])


def _make_spmm(nx):
    @functools.partial(
        pl.kernel,
        out_type=jax.ShapeDtypeStruct((nx, 2, _N_PAD, _D), jnp.float32),
        mesh=_mesh,
        scratch_types=[
            pltpu.VMEM((_HCHK, _K), jnp.int32),
            pltpu.VMEM((_HCHK, _K), jnp.int32),
            pltpu.VMEM((_K, _D), jnp.float32),
            pltpu.VMEM((_K, _D), jnp.float32),
            pltpu.VMEM((16, _D), jnp.float32),
            pltpu.VMEM_SHARED((_N_PAD, _D), jnp.float32),
            pltpu.SemaphoreType.DMA,
            pltpu.SemaphoreType.DMA,
            pltpu.SemaphoreType.DMA,
        ],
    )
    def _spmm(x_hbm, src_hbm, dst_hbm, z16_hbm, out_hbm,
              src_v, dst_v, rows0_v, rows1_v, zeros_v, agg_sh,
              gsem0, gsem1, zsem):
        c = lax.axis_index("c")
        s = lax.axis_index("s")
        pltpu.sync_copy(z16_hbm, zeros_v)
        @pl.loop(0, nx)
        def _pass(hx):
            @pl.loop(0, _RPS // 16)
            def _z(k):
                pltpu.async_copy(zeros_v,
                                 agg_sh.at[pl.ds(s * _RPS + k * 16, 16)], zsem)
            pltpu.sync_copy(src_hbm.at[c].at[s].at[0], src_v)
            pltpu.sync_copy(dst_hbm.at[c].at[s].at[0], dst_v)
            pltpu.make_async_copy(x_hbm.at[0].at[pl.ds(0, _RPS)],
                                  agg_sh.at[pl.ds(s * _RPS, _RPS)], zsem).wait()
            plsc.subcore_barrier()
            @pl.loop(0, 2)
            def _half(hh):
                @pl.when(hh == 1)
                def _():
                    pltpu.sync_copy(src_hbm.at[c].at[s].at[1], src_v)
                    pltpu.sync_copy(dst_hbm.at[c].at[s].at[1], dst_v)
                pltpu.async_copy(x_hbm.at[hx].at[src_v.at[0]], rows0_v, gsem0)
                pltpu.async_copy(x_hbm.at[hx].at[src_v.at[1]], rows1_v, gsem1)
                @pl.loop(0, _HCHK, step=2)
                def _e(j):
                    pltpu.make_async_copy(x_hbm.at[hx].at[src_v.at[j]],
                                          rows0_v, gsem0).wait()
                    pltpu.sync_copy(rows0_v, agg_sh.at[dst_v.at[j]], add=True)
                    @pl.when(j + 2 < _HCHK)
                    def _():
                        pltpu.async_copy(x_hbm.at[hx].at[src_v.at[j + 2]],
                                         rows0_v, gsem0)
                    pltpu.make_async_copy(x_hbm.at[hx].at[src_v.at[j + 1]],
                                          rows1_v, gsem1).wait()
                    pltpu.sync_copy(rows1_v, agg_sh.at[dst_v.at[j + 1]],
                                    add=True)
                    @pl.when(j + 3 < _HCHK)
                    def _():
                        pltpu.async_copy(x_hbm.at[hx].at[src_v.at[j + 3]],
                                         rows1_v, gsem1)
            plsc.subcore_barrier()
            pltpu.sync_copy(agg_sh.at[pl.ds(s * _RPS, _RPS)],
                            out_hbm.at[hx].at[c].at[pl.ds(s * _RPS, _RPS)])
    return _spmm


_sc_spmm1 = _make_spmm(1)
_sc_spmm2 = _make_spmm(2)


_R = 2048


def _prep_body(deg_ref, h_ref, invout_ref, invin_ref, xs_ref):
    d = deg_ref[...]
    io = lax.rsqrt(jnp.maximum(d[0, 0] + d[1, 0], 1.0))[:, None]
    ii = lax.rsqrt(jnp.maximum(d[0, 1] + d[1, 1], 1.0))[:, None]
    invout_ref[...] = io
    invin_ref[...] = ii
    xs_ref[...] = h_ref[...] * io


def _mid_body(pa_ref, invin_ref, invout_ref, w1_ref, b1_ref, h1_ref):
    agg = (pa_ref[0] + pa_ref[1]) * invin_ref[...]
    y = jnp.dot(agg, w1_ref[...], preferred_element_type=jnp.float32)
    y = jnp.maximum(y + b1_ref[...], 0.0) * invout_ref[...]
    h1_ref[0] = y[:, :_D]
    h1_ref[1] = y[:, _D:]


def _fin_body(p_ref, invin_ref, w2a_ref, w2b_ref, b2_ref,
              wc_ref, bc_ref, out_ref, pool_ref):
    i = pl.program_id(0)
    ii = invin_ref[...]
    agg_a = (p_ref[0, 0] + p_ref[0, 1]) * ii
    agg_b = (p_ref[1, 0] + p_ref[1, 1]) * ii
    y = (jnp.dot(agg_a, w2a_ref[...], preferred_element_type=jnp.float32)
         + jnp.dot(agg_b, w2b_ref[...], preferred_element_type=jnp.float32))
    y = jnp.maximum(y + b2_ref[...], 0.0)
    rows = lax.broadcasted_iota(jnp.int32, (_R, 1), 0) + i * _R
    y = jnp.where(rows < _N, y, 0.0)
    bm = jnp.max(y, axis=0, keepdims=True)
    @pl.when(i == 0)
    def _():
        pool_ref[...] = bm
    @pl.when(i > 0)
    def _():
        pool_ref[...] = jnp.maximum(pool_ref[...], bm)
    @pl.when(i == pl.num_programs(0) - 1)
    def _():
        out_ref[...] = (jnp.dot(pool_ref[...], wc_ref[...],
                                preferred_element_type=jnp.float32)
                        + bc_ref[...])


def _tc_prep(deg_parts, h_pad):
    g = _N_PAD // _R
    return pl.pallas_call(
        _prep_body,
        grid=(g,),
        in_specs=[
            pl.BlockSpec((2, 2, _R), lambda i: (0, 0, i)),
            pl.BlockSpec((_R, _D), lambda i: (i, 0)),
        ],
        out_specs=[
            pl.BlockSpec((_R, 1), lambda i: (i, 0)),
            pl.BlockSpec((_R, 1), lambda i: (i, 0)),
            pl.BlockSpec((_R, _D), lambda i: (i, 0)),
        ],
        out_shape=[
            jax.ShapeDtypeStruct((_N_PAD, 1), jnp.float32),
            jax.ShapeDtypeStruct((_N_PAD, 1), jnp.float32),
            jax.ShapeDtypeStruct((_N_PAD, _D), jnp.float32),
        ],
    )(deg_parts, h_pad)


def _tc_mid(parts, invin, invout, w1, b1r):
    g = _N_PAD // _R
    return pl.pallas_call(
        _mid_body,
        grid=(g,),
        in_specs=[
            pl.BlockSpec((2, _R, _D), lambda i: (0, i, 0)),
            pl.BlockSpec((_R, 1), lambda i: (i, 0)),
            pl.BlockSpec((_R, 1), lambda i: (i, 0)),
            pl.BlockSpec((_D, _H), lambda i: (0, 0)),
            pl.BlockSpec((1, _H), lambda i: (0, 0)),
        ],
        out_specs=[
            pl.BlockSpec((2, _R, _D), lambda i: (0, i, 0)),
        ],
        out_shape=[
            jax.ShapeDtypeStruct((2, _N_PAD, _D), jnp.float32),
        ],
    )(parts, invin, invout, w1, b1r)


def _tc_final(parts, invin, w2a, w2b, b2r, wc_pad, bc_pad):
    g = _N_PAD // _R
    return pl.pallas_call(
        _fin_body,
        grid=(g,),
        in_specs=[
            pl.BlockSpec((2, 2, _R, _D), lambda i: (0, 0, i, 0)),
            pl.BlockSpec((_R, 1), lambda i: (i, 0)),
            pl.BlockSpec((_D, _H), lambda i: (0, 0)),
            pl.BlockSpec((_D, _H), lambda i: (0, 0)),
            pl.BlockSpec((1, _H), lambda i: (0, 0)),
            pl.BlockSpec((_H, 128), lambda i: (0, 0)),
            pl.BlockSpec((1, 128), lambda i: (0, 0)),
        ],
        out_specs=pl.BlockSpec((1, 128), lambda i: (0, 0)),
        out_shape=jax.ShapeDtypeStruct((1, 128), jnp.float32),
        scratch_shapes=[pltpu.VMEM((1, _H), jnp.float32)],
    )(parts, invin, w2a, w2b, b2r, wc_pad, bc_pad)


def kernel(h, edge_index, W1, b1, W2, b2, Wc, bc):
    src = edge_index[0]
    dst = edge_index[1]
    ppt = _E_PAD // _NW - _E // _NW
    pads = jnp.broadcast_to(_N + jnp.arange(ppt, dtype=jnp.int32), (_NW, ppt))
    srcp = jnp.concatenate([src.reshape(_NW, _E // _NW), pads],
                           axis=1).reshape(2, 16, 2, _HCHK, _K)
    dstp = jnp.concatenate([dst.reshape(_NW, _E // _NW), pads],
                           axis=1).reshape(2, 16, 2, _HCHK, _K)

    ones128 = jnp.ones((_K,), jnp.float32)
    zeros128 = jnp.zeros((_K,), jnp.float32)
    z16 = jnp.zeros((16, _D), jnp.float32)

    deg_parts = _sc_degrees(srcp, dstp, ones128, zeros128)

    h_pad = jnp.pad(h, ((0, _N_PAD - _N), (0, 0)))
    invout, invin, xs = _tc_prep(deg_parts, h_pad)

    agg1_parts = _sc_spmm1(xs[None], srcp, dstp, z16)

    (h1,) = _tc_mid(agg1_parts[0], invin, invout, W1, b1.reshape(1, _H))

    agg2_parts = _sc_spmm2(h1, srcp, dstp, z16)

    wc_pad = jnp.pad(Wc, ((0, 0), (0, 128 - Wc.shape[1])))
    bc_pad = jnp.pad(bc, (0, 128 - bc.shape[0])).reshape(1, 128)
    out = _tc_final(agg2_parts, invin,
                    W2[:_D], W2[_D:], b2.reshape(1, _H), wc_pad, bc_pad)
    return out[0, :Wc.shape[1]]


# speedup vs baseline: 3.7523x; 1.0070x over previous
import functools

import jax
import jax.numpy as jnp
from jax import lax
from jax.experimental import pallas as pl
from jax.experimental.pallas import tpu as pltpu
from jax.experimental.pallas import tpu_sc as plsc

_N = 10000
_E = 320000
_D = 128
_H = 256

_NW = 32
_K = 128
_CHK = 80
_HCHK = _CHK // 2
_E_PAD = _NW * _K * _CHK
_N_PAD = 10240
_RPS = _N_PAD // 16

_mesh = plsc.VectorSubcoreMesh(core_axis_name="c", subcore_axis_name="s")


@functools.partial(
    pl.kernel,
    out_type=jax.ShapeDtypeStruct((2, 2, _N_PAD), jnp.float32),
    mesh=_mesh,
    scratch_types=[
        pltpu.VMEM((2, _HCHK, _K), jnp.int32),
        pltpu.VMEM((2, _HCHK, _K), jnp.int32),
        pltpu.VMEM((_K,), jnp.float32),
        pltpu.VMEM((_K,), jnp.float32),
        pltpu.VMEM_SHARED((_N_PAD,), jnp.float32),
        pltpu.VMEM_SHARED((_N_PAD,), jnp.float32),
        pltpu.SemaphoreType.DMA,
    ],
)
def _sc_degrees(src_hbm, dst_hbm, ones_hbm, zeros_hbm, out_hbm,
                src_v, dst_v, ones_v, zeros_v, dego_sh, degi_sh, dsem):
    c = lax.axis_index("c")
    s = lax.axis_index("s")
    pltpu.sync_copy(ones_hbm, ones_v)
    pltpu.sync_copy(zeros_hbm, zeros_v)
    @pl.loop(0, _RPS // _K)
    def _z(k):
        pltpu.sync_copy(zeros_v, dego_sh.at[pl.ds(s * _RPS + k * _K, _K)])
        pltpu.sync_copy(zeros_v, degi_sh.at[pl.ds(s * _RPS + k * _K, _K)])
    plsc.subcore_barrier()
    pltpu.sync_copy(src_hbm.at[c].at[s], src_v)
    pltpu.sync_copy(dst_hbm.at[c].at[s], dst_v)
    @pl.loop(0, 2)
    def _half(hh):
        @pl.loop(0, _HCHK)
        def _e(j):
            pltpu.async_copy(ones_v, dego_sh.at[src_v.at[hh].at[j]], dsem,
                             add=True)
            pltpu.async_copy(ones_v, degi_sh.at[dst_v.at[hh].at[j]], dsem,
                             add=True)
    @pl.loop(0, 4)
    def _drain(k):
        pltpu.make_async_copy(src_hbm.at[c].at[s].at[0], src_v.at[0],
                              dsem).wait()
    plsc.subcore_barrier()
    pltpu.sync_copy(dego_sh.at[pl.ds(s * _RPS, _RPS)],
                    out_hbm.at[c].at[0].at[pl.ds(s * _RPS, _RPS)])
    pltpu.sync_copy(degi_sh.at[pl.ds(s * _RPS, _RPS)],
                    out_hbm.at[c].at[1].at[pl.ds(s * _RPS, _RPS)])


def _make_spmm(nx):
    @functools.partial(
        pl.kernel,
        out_type=jax.ShapeDtypeStruct((nx, 2, _N_PAD, _D), jnp.float32),
        mesh=_mesh,
        scratch_types=[
            pltpu.VMEM((_HCHK, _K), jnp.int32),
            pltpu.VMEM((_HCHK, _K), jnp.int32),
            pltpu.VMEM((_K, _D), jnp.float32),
            pltpu.VMEM((_K, _D), jnp.float32),
            pltpu.VMEM((16, _D), jnp.float32),
            pltpu.VMEM_SHARED((_N_PAD, _D), jnp.float32),
            pltpu.SemaphoreType.DMA,
            pltpu.SemaphoreType.DMA,
            pltpu.SemaphoreType.DMA,
        ],
    )
    def _spmm(x_hbm, src_hbm, dst_hbm, z16_hbm, out_hbm,
              src_v, dst_v, rows0_v, rows1_v, zeros_v, agg_sh,
              gsem0, gsem1, zsem):
        c = lax.axis_index("c")
        s = lax.axis_index("s")
        pltpu.sync_copy(z16_hbm, zeros_v)
        @pl.loop(0, nx)
        def _pass(hx):
            @pl.loop(0, _RPS // 16)
            def _z(k):
                pltpu.async_copy(zeros_v,
                                 agg_sh.at[pl.ds(s * _RPS + k * 16, 16)], zsem)
            pltpu.sync_copy(src_hbm.at[c].at[s].at[0], src_v)
            pltpu.sync_copy(dst_hbm.at[c].at[s].at[0], dst_v)
            pltpu.async_copy(x_hbm.at[hx].at[src_v.at[0]], rows0_v, gsem0)
            pltpu.async_copy(x_hbm.at[hx].at[src_v.at[1]], rows1_v, gsem1)
            pltpu.make_async_copy(x_hbm.at[0].at[pl.ds(0, _RPS)],
                                  agg_sh.at[pl.ds(s * _RPS, _RPS)], zsem).wait()
            plsc.subcore_barrier()
            @pl.loop(0, 2)
            def _half(hh):
                @pl.when(hh == 1)
                def _():
                    pltpu.sync_copy(src_hbm.at[c].at[s].at[1], src_v)
                    pltpu.sync_copy(dst_hbm.at[c].at[s].at[1], dst_v)
                    pltpu.async_copy(x_hbm.at[hx].at[src_v.at[0]],
                                     rows0_v, gsem0)
                    pltpu.async_copy(x_hbm.at[hx].at[src_v.at[1]],
                                     rows1_v, gsem1)
                @pl.loop(0, _HCHK, step=2)
                def _e(j):
                    pltpu.make_async_copy(x_hbm.at[hx].at[src_v.at[j]],
                                          rows0_v, gsem0).wait()
                    pltpu.sync_copy(rows0_v, agg_sh.at[dst_v.at[j]], add=True)
                    @pl.when(j + 2 < _HCHK)
                    def _():
                        pltpu.async_copy(x_hbm.at[hx].at[src_v.at[j + 2]],
                                         rows0_v, gsem0)
                    pltpu.make_async_copy(x_hbm.at[hx].at[src_v.at[j + 1]],
                                          rows1_v, gsem1).wait()
                    pltpu.sync_copy(rows1_v, agg_sh.at[dst_v.at[j + 1]],
                                    add=True)
                    @pl.when(j + 3 < _HCHK)
                    def _():
                        pltpu.async_copy(x_hbm.at[hx].at[src_v.at[j + 3]],
                                         rows1_v, gsem1)
            plsc.subcore_barrier()
            pltpu.sync_copy(agg_sh.at[pl.ds(s * _RPS, _RPS)],
                            out_hbm.at[hx].at[c].at[pl.ds(s * _RPS, _RPS)])
    return _spmm


_sc_spmm1 = _make_spmm(1)
_sc_spmm2 = _make_spmm(2)


_R = 2048


def _prep_body(deg_ref, h_ref, invout_ref, invin_ref, xs_ref):
    d = deg_ref[...]
    io = lax.rsqrt(jnp.maximum(d[0, 0] + d[1, 0], 1.0))[:, None]
    ii = lax.rsqrt(jnp.maximum(d[0, 1] + d[1, 1], 1.0))[:, None]
    invout_ref[...] = io
    invin_ref[...] = ii
    xs_ref[...] = h_ref[...] * io


def _mid_body(pa_ref, invin_ref, invout_ref, w1_ref, b1_ref, h1_ref):
    agg = (pa_ref[0] + pa_ref[1]) * invin_ref[...]
    y = jnp.dot(agg, w1_ref[...], preferred_element_type=jnp.float32)
    y = jnp.maximum(y + b1_ref[...], 0.0) * invout_ref[...]
    h1_ref[0] = y[:, :_D]
    h1_ref[1] = y[:, _D:]


def _fin_body(p_ref, invin_ref, w2a_ref, w2b_ref, b2_ref,
              wc_ref, bc_ref, out_ref, pool_ref):
    i = pl.program_id(0)
    ii = invin_ref[...]
    agg_a = (p_ref[0, 0] + p_ref[0, 1]) * ii
    agg_b = (p_ref[1, 0] + p_ref[1, 1]) * ii
    y = (jnp.dot(agg_a, w2a_ref[...], preferred_element_type=jnp.float32)
         + jnp.dot(agg_b, w2b_ref[...], preferred_element_type=jnp.float32))
    y = jnp.maximum(y + b2_ref[...], 0.0)
    rows = lax.broadcasted_iota(jnp.int32, (_R, 1), 0) + i * _R
    y = jnp.where(rows < _N, y, 0.0)
    bm = jnp.max(y, axis=0, keepdims=True)
    @pl.when(i == 0)
    def _():
        pool_ref[...] = bm
    @pl.when(i > 0)
    def _():
        pool_ref[...] = jnp.maximum(pool_ref[...], bm)
    @pl.when(i == pl.num_programs(0) - 1)
    def _():
        out_ref[...] = (jnp.dot(pool_ref[...], wc_ref[...],
                                preferred_element_type=jnp.float32)
                        + bc_ref[...])


def _tc_prep(deg_parts, h_pad):
    g = _N_PAD // _R
    return pl.pallas_call(
        _prep_body,
        grid=(g,),
        in_specs=[
            pl.BlockSpec((2, 2, _R), lambda i: (0, 0, i)),
            pl.BlockSpec((_R, _D), lambda i: (i, 0)),
        ],
        out_specs=[
            pl.BlockSpec((_R, 1), lambda i: (i, 0)),
            pl.BlockSpec((_R, 1), lambda i: (i, 0)),
            pl.BlockSpec((_R, _D), lambda i: (i, 0)),
        ],
        out_shape=[
            jax.ShapeDtypeStruct((_N_PAD, 1), jnp.float32),
            jax.ShapeDtypeStruct((_N_PAD, 1), jnp.float32),
            jax.ShapeDtypeStruct((_N_PAD, _D), jnp.float32),
        ],
    )(deg_parts, h_pad)


def _tc_mid(parts, invin, invout, w1, b1r):
    g = _N_PAD // _R
    return pl.pallas_call(
        _mid_body,
        grid=(g,),
        in_specs=[
            pl.BlockSpec((2, _R, _D), lambda i: (0, i, 0)),
            pl.BlockSpec((_R, 1), lambda i: (i, 0)),
            pl.BlockSpec((_R, 1), lambda i: (i, 0)),
            pl.BlockSpec((_D, _H), lambda i: (0, 0)),
            pl.BlockSpec((1, _H), lambda i: (0, 0)),
        ],
        out_specs=[
            pl.BlockSpec((2, _R, _D), lambda i: (0, i, 0)),
        ],
        out_shape=[
            jax.ShapeDtypeStruct((2, _N_PAD, _D), jnp.float32),
        ],
    )(parts, invin, invout, w1, b1r)


def _tc_final(parts, invin, w2a, w2b, b2r, wc_pad, bc_pad):
    g = _N_PAD // _R
    return pl.pallas_call(
        _fin_body,
        grid=(g,),
        in_specs=[
            pl.BlockSpec((2, 2, _R, _D), lambda i: (0, 0, i, 0)),
            pl.BlockSpec((_R, 1), lambda i: (i, 0)),
            pl.BlockSpec((_D, _H), lambda i: (0, 0)),
            pl.BlockSpec((_D, _H), lambda i: (0, 0)),
            pl.BlockSpec((1, _H), lambda i: (0, 0)),
            pl.BlockSpec((_H, 128), lambda i: (0, 0)),
            pl.BlockSpec((1, 128), lambda i: (0, 0)),
        ],
        out_specs=pl.BlockSpec((1, 128), lambda i: (0, 0)),
        out_shape=jax.ShapeDtypeStruct((1, 128), jnp.float32),
        scratch_shapes=[pltpu.VMEM((1, _H), jnp.float32)],
    )(parts, invin, w2a, w2b, b2r, wc_pad, bc_pad)


def kernel(h, edge_index, W1, b1, W2, b2, Wc, bc):
    src = edge_index[0]
    dst = edge_index[1]
    ppt = _E_PAD // _NW - _E // _NW
    pads = jnp.broadcast_to(_N + jnp.arange(ppt, dtype=jnp.int32), (_NW, ppt))
    srcp = jnp.concatenate([src.reshape(_NW, _E // _NW), pads],
                           axis=1).reshape(2, 16, 2, _HCHK, _K)
    dstp = jnp.concatenate([dst.reshape(_NW, _E // _NW), pads],
                           axis=1).reshape(2, 16, 2, _HCHK, _K)

    ones128 = jnp.ones((_K,), jnp.float32)
    zeros128 = jnp.zeros((_K,), jnp.float32)
    z16 = jnp.zeros((16, _D), jnp.float32)

    deg_parts = _sc_degrees(srcp, dstp, ones128, zeros128)

    h_pad = jnp.pad(h, ((0, _N_PAD - _N), (0, 0)))
    invout, invin, xs = _tc_prep(deg_parts, h_pad)

    agg1_parts = _sc_spmm1(xs[None], srcp, dstp, z16)

    (h1,) = _tc_mid(agg1_parts[0], invin, invout, W1, b1.reshape(1, _H))

    agg2_parts = _sc_spmm2(h1, srcp, dstp, z16)

    wc_pad = jnp.pad(Wc, ((0, 0), (0, 128 - Wc.shape[1])))
    bc_pad = jnp.pad(bc, (0, 128 - bc.shape[0])).reshape(1, 128)
    out = _tc_final(agg2_parts, invin,
                    W2[:_D], W2[_D:], b2.reshape(1, _H), wc_pad, bc_pad)
    return out[0, :Wc.shape[1]]


# speedup vs baseline: 3.7590x; 1.0018x over previous
import functools

import jax
import jax.numpy as jnp
from jax import lax
from jax.experimental import pallas as pl
from jax.experimental.pallas import tpu as pltpu
from jax.experimental.pallas import tpu_sc as plsc

_N = 10000
_E = 320000
_D = 128
_H = 256

_NW = 32
_K = 128
_CHK = 80
_HCHK = _CHK // 2
_E_PAD = _NW * _K * _CHK
_N_PAD = 10240
_RPS = _N_PAD // 16

_mesh = plsc.VectorSubcoreMesh(core_axis_name="c", subcore_axis_name="s")


@functools.partial(
    pl.kernel,
    out_type=jax.ShapeDtypeStruct((2, 2, _N_PAD), jnp.float32),
    mesh=_mesh,
    scratch_types=[
        pltpu.VMEM((2, _HCHK, _K), jnp.int32),
        pltpu.VMEM((2, _HCHK, _K), jnp.int32),
        pltpu.VMEM((_K,), jnp.float32),
        pltpu.VMEM((_K,), jnp.float32),
        pltpu.VMEM_SHARED((_N_PAD,), jnp.float32),
        pltpu.VMEM_SHARED((_N_PAD,), jnp.float32),
        pltpu.SemaphoreType.DMA,
    ],
)
def _sc_degrees(src_hbm, dst_hbm, ones_hbm, zeros_hbm, out_hbm,
                src_v, dst_v, ones_v, zeros_v, dego_sh, degi_sh, dsem):
    c = lax.axis_index("c")
    s = lax.axis_index("s")
    pltpu.sync_copy(ones_hbm, ones_v)
    pltpu.sync_copy(zeros_hbm, zeros_v)
    @pl.loop(0, _RPS // _K)
    def _z(k):
        pltpu.async_copy(zeros_v, dego_sh.at[pl.ds(s * _RPS + k * _K, _K)],
                         dsem)
        pltpu.async_copy(zeros_v, degi_sh.at[pl.ds(s * _RPS + k * _K, _K)],
                         dsem)
    pltpu.sync_copy(src_hbm.at[c].at[s], src_v)
    pltpu.sync_copy(dst_hbm.at[c].at[s], dst_v)
    @pl.loop(0, 2 * (_RPS // _K))
    def _zdrain(k):
        pltpu.make_async_copy(ones_hbm, zeros_v, dsem).wait()
    plsc.subcore_barrier()
    @pl.loop(0, 2)
    def _half(hh):
        @pl.loop(0, _HCHK)
        def _e(j):
            pltpu.async_copy(ones_v, dego_sh.at[src_v.at[hh].at[j]], dsem,
                             add=True)
            pltpu.async_copy(ones_v, degi_sh.at[dst_v.at[hh].at[j]], dsem,
                             add=True)
    @pl.loop(0, 4)
    def _drain(k):
        pltpu.make_async_copy(src_hbm.at[c].at[s].at[0], src_v.at[0],
                              dsem).wait()
    plsc.subcore_barrier()
    pltpu.sync_copy(dego_sh.at[pl.ds(s * _RPS, _RPS)],
                    out_hbm.at[c].at[0].at[pl.ds(s * _RPS, _RPS)])
    pltpu.sync_copy(degi_sh.at[pl.ds(s * _RPS, _RPS)],
                    out_hbm.at[c].at[1].at[pl.ds(s * _RPS, _RPS)])


def _make_spmm(nx):
    @functools.partial(
        pl.kernel,
        out_type=jax.ShapeDtypeStruct((nx, 2, _N_PAD, _D), jnp.float32),
        mesh=_mesh,
        scratch_types=[
            pltpu.VMEM((_HCHK, _K), jnp.int32),
            pltpu.VMEM((_HCHK, _K), jnp.int32),
            pltpu.VMEM((_K, _D), jnp.float32),
            pltpu.VMEM((_K, _D), jnp.float32),
            pltpu.VMEM((16, _D), jnp.float32),
            pltpu.VMEM_SHARED((_N_PAD, _D), jnp.float32),
            pltpu.SemaphoreType.DMA,
            pltpu.SemaphoreType.DMA,
            pltpu.SemaphoreType.DMA,
        ],
    )
    def _spmm(x_hbm, src_hbm, dst_hbm, z16_hbm, out_hbm,
              src_v, dst_v, rows0_v, rows1_v, zeros_v, agg_sh,
              gsem0, gsem1, zsem):
        c = lax.axis_index("c")
        s = lax.axis_index("s")
        pltpu.sync_copy(z16_hbm, zeros_v)
        @pl.loop(0, nx)
        def _pass(hx):
            @pl.loop(0, _RPS // 16)
            def _z(k):
                pltpu.async_copy(zeros_v,
                                 agg_sh.at[pl.ds(s * _RPS + k * 16, 16)], zsem)
            pltpu.sync_copy(src_hbm.at[c].at[s].at[0], src_v)
            pltpu.sync_copy(dst_hbm.at[c].at[s].at[0], dst_v)
            pltpu.async_copy(x_hbm.at[hx].at[src_v.at[0]], rows0_v, gsem0)
            pltpu.async_copy(x_hbm.at[hx].at[src_v.at[1]], rows1_v, gsem1)
            pltpu.make_async_copy(x_hbm.at[0].at[pl.ds(0, _RPS)],
                                  agg_sh.at[pl.ds(s * _RPS, _RPS)], zsem).wait()
            plsc.subcore_barrier()
            @pl.loop(0, 2)
            def _half(hh):
                @pl.when(hh == 1)
                def _():
                    pltpu.sync_copy(src_hbm.at[c].at[s].at[1], src_v)
                    pltpu.sync_copy(dst_hbm.at[c].at[s].at[1], dst_v)
                    pltpu.async_copy(x_hbm.at[hx].at[src_v.at[0]],
                                     rows0_v, gsem0)
                    pltpu.async_copy(x_hbm.at[hx].at[src_v.at[1]],
                                     rows1_v, gsem1)
                @pl.loop(0, _HCHK, step=2)
                def _e(j):
                    pltpu.make_async_copy(x_hbm.at[hx].at[src_v.at[j]],
                                          rows0_v, gsem0).wait()
                    pltpu.sync_copy(rows0_v, agg_sh.at[dst_v.at[j]], add=True)
                    @pl.when(j + 2 < _HCHK)
                    def _():
                        pltpu.async_copy(x_hbm.at[hx].at[src_v.at[j + 2]],
                                         rows0_v, gsem0)
                    pltpu.make_async_copy(x_hbm.at[hx].at[src_v.at[j + 1]],
                                          rows1_v, gsem1).wait()
                    pltpu.sync_copy(rows1_v, agg_sh.at[dst_v.at[j + 1]],
                                    add=True)
                    @pl.when(j + 3 < _HCHK)
                    def _():
                        pltpu.async_copy(x_hbm.at[hx].at[src_v.at[j + 3]],
                                         rows1_v, gsem1)
            plsc.subcore_barrier()
            pltpu.sync_copy(agg_sh.at[pl.ds(s * _RPS, _RPS)],
                            out_hbm.at[hx].at[c].at[pl.ds(s * _RPS, _RPS)])
    return _spmm


_sc_spmm1 = _make_spmm(1)
_sc_spmm2 = _make_spmm(2)


_R = 2048


def _prep_body(deg_ref, h_ref, invout_ref, invin_ref, xs_ref):
    d = deg_ref[...]
    io = lax.rsqrt(jnp.maximum(d[0, 0] + d[1, 0], 1.0))[:, None]
    ii = lax.rsqrt(jnp.maximum(d[0, 1] + d[1, 1], 1.0))[:, None]
    invout_ref[...] = io
    invin_ref[...] = ii
    xs_ref[...] = h_ref[...] * io


def _mid_body(pa_ref, invin_ref, invout_ref, w1_ref, b1_ref, h1_ref):
    agg = (pa_ref[0] + pa_ref[1]) * invin_ref[...]
    y = jnp.dot(agg, w1_ref[...], preferred_element_type=jnp.float32)
    y = jnp.maximum(y + b1_ref[...], 0.0) * invout_ref[...]
    h1_ref[0] = y[:, :_D]
    h1_ref[1] = y[:, _D:]


def _fin_body(p_ref, invin_ref, w2a_ref, w2b_ref, b2_ref,
              wc_ref, bc_ref, out_ref, pool_ref):
    i = pl.program_id(0)
    ii = invin_ref[...]
    agg_a = (p_ref[0, 0] + p_ref[0, 1]) * ii
    agg_b = (p_ref[1, 0] + p_ref[1, 1]) * ii
    y = (jnp.dot(agg_a, w2a_ref[...], preferred_element_type=jnp.float32)
         + jnp.dot(agg_b, w2b_ref[...], preferred_element_type=jnp.float32))
    y = jnp.maximum(y + b2_ref[...], 0.0)
    rows = lax.broadcasted_iota(jnp.int32, (_R, 1), 0) + i * _R
    y = jnp.where(rows < _N, y, 0.0)
    bm = jnp.max(y, axis=0, keepdims=True)
    @pl.when(i == 0)
    def _():
        pool_ref[...] = bm
    @pl.when(i > 0)
    def _():
        pool_ref[...] = jnp.maximum(pool_ref[...], bm)
    @pl.when(i == pl.num_programs(0) - 1)
    def _():
        out_ref[...] = (jnp.dot(pool_ref[...], wc_ref[...],
                                preferred_element_type=jnp.float32)
                        + bc_ref[...])


def _tc_prep(deg_parts, h_pad):
    g = _N_PAD // _R
    return pl.pallas_call(
        _prep_body,
        grid=(g,),
        in_specs=[
            pl.BlockSpec((2, 2, _R), lambda i: (0, 0, i)),
            pl.BlockSpec((_R, _D), lambda i: (i, 0)),
        ],
        out_specs=[
            pl.BlockSpec((_R, 1), lambda i: (i, 0)),
            pl.BlockSpec((_R, 1), lambda i: (i, 0)),
            pl.BlockSpec((_R, _D), lambda i: (i, 0)),
        ],
        out_shape=[
            jax.ShapeDtypeStruct((_N_PAD, 1), jnp.float32),
            jax.ShapeDtypeStruct((_N_PAD, 1), jnp.float32),
            jax.ShapeDtypeStruct((_N_PAD, _D), jnp.float32),
        ],
    )(deg_parts, h_pad)


def _tc_mid(parts, invin, invout, w1, b1r):
    g = _N_PAD // _R
    return pl.pallas_call(
        _mid_body,
        grid=(g,),
        in_specs=[
            pl.BlockSpec((2, _R, _D), lambda i: (0, i, 0)),
            pl.BlockSpec((_R, 1), lambda i: (i, 0)),
            pl.BlockSpec((_R, 1), lambda i: (i, 0)),
            pl.BlockSpec((_D, _H), lambda i: (0, 0)),
            pl.BlockSpec((1, _H), lambda i: (0, 0)),
        ],
        out_specs=[
            pl.BlockSpec((2, _R, _D), lambda i: (0, i, 0)),
        ],
        out_shape=[
            jax.ShapeDtypeStruct((2, _N_PAD, _D), jnp.float32),
        ],
    )(parts, invin, invout, w1, b1r)


def _tc_final(parts, invin, w2a, w2b, b2r, wc_pad, bc_pad):
    g = _N_PAD // _R
    return pl.pallas_call(
        _fin_body,
        grid=(g,),
        in_specs=[
            pl.BlockSpec((2, 2, _R, _D), lambda i: (0, 0, i, 0)),
            pl.BlockSpec((_R, 1), lambda i: (i, 0)),
            pl.BlockSpec((_D, _H), lambda i: (0, 0)),
            pl.BlockSpec((_D, _H), lambda i: (0, 0)),
            pl.BlockSpec((1, _H), lambda i: (0, 0)),
            pl.BlockSpec((_H, 128), lambda i: (0, 0)),
            pl.BlockSpec((1, 128), lambda i: (0, 0)),
        ],
        out_specs=pl.BlockSpec((1, 128), lambda i: (0, 0)),
        out_shape=jax.ShapeDtypeStruct((1, 128), jnp.float32),
        scratch_shapes=[pltpu.VMEM((1, _H), jnp.float32)],
    )(parts, invin, w2a, w2b, b2r, wc_pad, bc_pad)


def kernel(h, edge_index, W1, b1, W2, b2, Wc, bc):
    src = edge_index[0]
    dst = edge_index[1]
    ppt = _E_PAD // _NW - _E // _NW
    pads = jnp.broadcast_to(_N + jnp.arange(ppt, dtype=jnp.int32), (_NW, ppt))
    srcp = jnp.concatenate([src.reshape(_NW, _E // _NW), pads],
                           axis=1).reshape(2, 16, 2, _HCHK, _K)
    dstp = jnp.concatenate([dst.reshape(_NW, _E // _NW), pads],
                           axis=1).reshape(2, 16, 2, _HCHK, _K)

    ones128 = jnp.ones((_K,), jnp.float32)
    zeros128 = jnp.zeros((_K,), jnp.float32)
    z16 = jnp.zeros((16, _D), jnp.float32)

    deg_parts = _sc_degrees(srcp, dstp, ones128, zeros128)

    h_pad = jnp.pad(h, ((0, _N_PAD - _N), (0, 0)))
    invout, invin, xs = _tc_prep(deg_parts, h_pad)

    agg1_parts = _sc_spmm1(xs[None], srcp, dstp, z16)

    (h1,) = _tc_mid(agg1_parts[0], invin, invout, W1, b1.reshape(1, _H))

    agg2_parts = _sc_spmm2(h1, srcp, dstp, z16)

    wc_pad = jnp.pad(Wc, ((0, 0), (0, 128 - Wc.shape[1])))
    bc_pad = jnp.pad(bc, (0, 128 - bc.shape[0])).reshape(1, 128)
    out = _tc_final(agg2_parts, invin,
                    W2[:_D], W2[_D:], b2.reshape(1, _H), wc_pad, bc_pad)
    return out[0, :Wc.shape[1]]
